# baseline (speedup 1.0000x reference)
import functools

import jax
import jax.numpy as jnp
from jax import lax
from jax.experimental import pallas as pl
from jax.experimental.pallas import tpu as pltpu
from jax.experimental.pallas import tpu_sc as plsc

B, S, H = 4, 1024, 1024
L = 16
NW = 32
TPW = (B * S) // NW
WPR = S // TPW
CHUNK = 16
NCH = TPW // CHUNK
GPH = H // L
U1 = 8
U2 = 4
NSLOT = 3
POSR = 1030
PTAB_R = 1040
WIN = 24
PAD_SLOT = WIN


def _fuse_tables_body(p_ref, t_ref, o_ref):
    o_ref[pl.ds(0, POSR), :] = p_ref[...] + t_ref[0, :][None, :]
    o_ref[pl.ds(POSR, PTAB_R - POSR), :] = jnp.zeros(
        (PTAB_R - POSR, p_ref.shape[1]), jnp.float32
    )


def _fuse_tables(pos_tab, type_tab):
    return pl.pallas_call(
        _fuse_tables_body,
        out_shape=jax.ShapeDtypeStruct((PTAB_R, pos_tab.shape[1]), jnp.float32),
    )(pos_tab, type_tab)


_mesh = plsc.VectorSubcoreMesh(
    core_axis_name="c", subcore_axis_name="s", num_cores=2, num_subcores=16
)


@functools.partial(
    pl.kernel,
    out_type=jax.ShapeDtypeStruct((B * S, H), jnp.float32),
    mesh=_mesh,
    scratch_types=[
        pltpu.VMEM((S,), jnp.int32),
        pltpu.VMEM((NCH, CHUNK), jnp.int32),
        pltpu.SMEM((TPW,), jnp.int32),
        pltpu.SMEM((NCH,), jnp.int32),
        pltpu.VMEM((NSLOT, CHUNK, H), jnp.float32),
        pltpu.VMEM((2, WIN + 1, H), jnp.float32),
        pltpu.VMEM((4, H), jnp.float32),
        pltpu.VMEM((H,), jnp.float32),
        pltpu.VMEM((H,), jnp.float32),
        [pltpu.SemaphoreType.DMA] * NSLOT,
        [pltpu.SemaphoreType.DMA] * 2,
        [pltpu.SemaphoreType.DMA] * NSLOT,
    ],
    compiler_params=pltpu.CompilerParams(needs_layout_passes=False),
)
def _sc_embed_ln(tok_hbm, word_hbm, ptab_hbm, gam_hbm, bet_hbm, out_hbm,
                 row_v, tokidx, lidx_v, scpref, wbuf, pbuf, obuf,
                 gam_v, bet_v, semw, semp, semo):
    cid = lax.axis_index("c")
    sid = lax.axis_index("s")
    wid = sid * 2 + cid
    brow = wid // WPR
    cpos = wid % WPR

    pltpu.sync_copy(tok_hbm.at[brow], row_v)
    pltpu.sync_copy(gam_hbm, gam_v)
    pltpu.sync_copy(bet_hbm, bet_v)
    pltpu.sync_copy(ptab_hbm.at[pl.ds(1, 1)], pbuf.at[0, pl.ds(PAD_SLOT, 1)])
    pltpu.sync_copy(ptab_hbm.at[pl.ds(1, 1)], pbuf.at[1, pl.ds(PAD_SLOT, 1)])

    def _pref(g, acc):
        v = row_v[pl.ds(g * L, L)]
        return acc + jnp.sum((v != 1).astype(jnp.int32))

    prefix = lax.fori_loop(0, cpos * (TPW // L), _pref, jnp.int32(0))

    def _pos(k, pref):
        scpref[k] = pref
        v = row_v[pl.ds(cpos * TPW + k * L, L)]
        mi = (v != 1).astype(jnp.int32)
        pc = plsc.cumsum(mi)
        tokidx[k, pl.ds(0, CHUNK)] = v
        d = (pref + 2) & 7
        lv = (pc - 1 + d) * mi + (1 - mi) * PAD_SLOT
        for i in range(L):
            lidx_v[k * L + i] = lv[i]
        return pref + jnp.sum(mi)

    lax.fori_loop(0, NCH, _pos, prefix)

    base = wid * TPW

    def _start_word(j, s):
        return pltpu.async_copy(word_hbm.at[tokidx.at[j]], wbuf.at[s], semw[s])

    def _start_pos(j, sp):
        a = pl.multiple_of(((scpref[j] + 2) >> 3) << 3, 8)
        return pltpu.async_copy(
            ptab_hbm.at[pl.ds(a, WIN)], pbuf.at[sp, pl.ds(0, WIN)], semp[sp]
        )

    def _compute(j, s, sp):
        def _blk(blk, _):
            bt = blk * 4
            mvs = []
            rvs = []
            for tt in range(4):
                r = lidx_v[j * CHUNK + bt + tt]
                z = jnp.zeros((L,), jnp.float32)

                @plsc.parallel_loop(0, GPH, unroll=U1, carry=(z, z))
                def _g1(g, carry, tt=tt, r=r):
                    acc, acc2 = carry
                    sl = pl.ds(g * L, L)
                    v = wbuf[s, bt + tt, sl] + pbuf[sp, r, sl]
                    obuf[tt, sl] = v
                    return acc + v, acc2 + v * v

                acc, acc2 = _g1
                meanv = jnp.broadcast_to(jnp.sum(acc) * (1.0 / H), (L,))
                msqv = jnp.broadcast_to(jnp.sum(acc2) * (1.0 / H), (L,))
                varv = msqv - meanv * meanv + 1e-5
                y = plsc.bitcast(
                    jnp.int32(0x5F3759DF) - (plsc.bitcast(varv, jnp.int32) >> 1),
                    jnp.float32,
                )
                y = y * (1.5 - 0.5 * varv * y * y)
                y = y * (1.5 - 0.5 * varv * y * y)
                y = y * (1.5 - 0.5 * varv * y * y)
                mvs.append(meanv)
                rvs.append(y)

            @plsc.parallel_loop(0, GPH, unroll=U2)
            def _g2(g):
                sl = pl.ds(g * L, L)
                gv = gam_v[sl]
                bv = bet_v[sl]
                for tt in range(4):
                    v = obuf[tt, sl]
                    wbuf[s, bt + tt, sl] = (v - mvs[tt]) * rvs[tt] * gv + bv

            return 0

        lax.fori_loop(0, CHUNK // 4, _blk, 0)

    wcopies = {}
    pcopies = {}
    writes = {}
    wcopies[0] = _start_word(0, 0)
    wcopies[1] = _start_word(1, 1)
    pcopies[0] = _start_pos(0, 0)
    for j in range(NCH):
        s = j % NSLOT
        sp = j % 2
        if j + 2 < NCH:
            if j >= 1:
                writes[j - 1].wait()
            wcopies[j + 2] = _start_word(j + 2, (j + 2) % NSLOT)
        if j + 1 < NCH:
            pcopies[j + 1] = _start_pos(j + 1, (j + 1) % 2)
        wcopies.pop(j).wait()
        pcopies.pop(j).wait()
        _compute(j, s, sp)
        writes[j] = pltpu.async_copy(
            wbuf.at[s], out_hbm.at[pl.ds(base + j * CHUNK, CHUNK)], semo[s]
        )
    for j in range(NCH - NSLOT, NCH):
        writes[j].wait()


def kernel(arg0_1, arg1_1, arg2_1, arg3_1, arg4_1, arg5_1):
    tok = arg0_1.astype(jnp.int32)
    ptab = _fuse_tables(arg5_1, arg2_1)
    flat = _sc_embed_ln(tok, arg1_1, ptab, arg3_1, arg4_1)
    out = flat.reshape(B, S, H)
    sel = jnp.full((B, S), -0.0, dtype=jnp.float32)
    return (out, sel)


# speedup vs baseline: 1.0722x; 1.0722x over previous
import functools

import jax
import jax.numpy as jnp
from jax import lax
from jax.experimental import pallas as pl
from jax.experimental.pallas import tpu as pltpu
from jax.experimental.pallas import tpu_sc as plsc

B, S, H = 4, 1024, 1024
L = 16
NW = 32
TPW = (B * S) // NW
WPR = S // TPW
CHUNK = 16
NCH = TPW // CHUNK
GPH = H // L
U1 = 8
U2 = 4
NSLOT = 3
POSR = 1030
PTAB_R = 1040
WIN = 24
PAD_SLOT = WIN


def _fuse_tables_body(p_ref, t_ref, o_ref):
    o_ref[pl.ds(0, POSR), :] = p_ref[...] + t_ref[0, :][None, :]
    o_ref[pl.ds(POSR, PTAB_R - POSR), :] = jnp.zeros(
        (PTAB_R - POSR, p_ref.shape[1]), jnp.float32
    )


def _fuse_tables(pos_tab, type_tab):
    return pl.pallas_call(
        _fuse_tables_body,
        out_shape=jax.ShapeDtypeStruct((PTAB_R, pos_tab.shape[1]), jnp.float32),
    )(pos_tab, type_tab)


_mesh = plsc.VectorSubcoreMesh(
    core_axis_name="c", subcore_axis_name="s", num_cores=2, num_subcores=16
)


@functools.partial(
    pl.kernel,
    out_type=jax.ShapeDtypeStruct((B * S, H), jnp.float32),
    mesh=_mesh,
    scratch_types=[
        pltpu.VMEM((S,), jnp.int32),
        pltpu.VMEM((NCH, CHUNK), jnp.int32),
        pltpu.SMEM((TPW,), jnp.int32),
        pltpu.SMEM((NCH,), jnp.int32),
        pltpu.VMEM((NSLOT, CHUNK, H), jnp.float32),
        pltpu.VMEM((2, WIN + 1, H), jnp.float32),
        pltpu.VMEM((4, H), jnp.float32),
        pltpu.VMEM((H,), jnp.float32),
        pltpu.VMEM((H,), jnp.float32),
        [pltpu.SemaphoreType.DMA] * NSLOT,
        [pltpu.SemaphoreType.DMA] * 2,
        [pltpu.SemaphoreType.DMA] * NSLOT,
    ],
    compiler_params=pltpu.CompilerParams(needs_layout_passes=False),
)
def _sc_embed_ln(tok_hbm, word_hbm, ptab_hbm, gam_hbm, bet_hbm, out_hbm,
                 row_v, tokidx, lidx_v, scpref, wbuf, pbuf, obuf,
                 gam_v, bet_v, semw, semp, semo):
    cid = lax.axis_index("c")
    sid = lax.axis_index("s")
    wid = sid * 2 + cid
    brow = wid // WPR
    cpos = wid % WPR

    pltpu.sync_copy(tok_hbm.at[brow], row_v)
    pltpu.sync_copy(gam_hbm, gam_v)
    pltpu.sync_copy(bet_hbm, bet_v)
    pltpu.sync_copy(ptab_hbm.at[pl.ds(1, 1)], pbuf.at[0, pl.ds(PAD_SLOT, 1)])
    pltpu.sync_copy(ptab_hbm.at[pl.ds(1, 1)], pbuf.at[1, pl.ds(PAD_SLOT, 1)])

    def _pref(g, acc):
        v = row_v[pl.ds(g * L, L)]
        return acc + jnp.sum((v != 1).astype(jnp.int32))

    prefix = lax.fori_loop(0, cpos * (TPW // L), _pref, jnp.int32(0))

    def _pos(k, pref):
        scpref[k] = pref
        v = row_v[pl.ds(cpos * TPW + k * L, L)]
        mi = (v != 1).astype(jnp.int32)
        pc = plsc.cumsum(mi)
        tokidx[k, pl.ds(0, CHUNK)] = v
        d = (pref + 2) & 7
        lv = (pc - 1 + d) * mi + (1 - mi) * PAD_SLOT
        for i in range(L):
            lidx_v[k * L + i] = lv[i]
        return pref + jnp.sum(mi)

    lax.fori_loop(0, NCH, _pos, prefix)

    base = wid * TPW

    def _start_word(j, s):
        return pltpu.async_copy(word_hbm.at[tokidx.at[j]], wbuf.at[s], semw[s])

    def _start_pos(j, sp):
        a = pl.multiple_of(((scpref[j] + 2) >> 3) << 3, 8)
        return pltpu.async_copy(
            ptab_hbm.at[pl.ds(a, WIN)], pbuf.at[sp, pl.ds(0, WIN)], semp[sp]
        )

    def _compute(j, s, sp):
        def _blk(blk, _):
            bt = blk * 4
            mvs = []
            rvs = []
            for tt in range(4):
                r = lidx_v[j * CHUNK + bt + tt]
                z = jnp.zeros((L,), jnp.float32)

                @plsc.parallel_loop(0, GPH, unroll=U1, carry=(z, z))
                def _g1(g, carry, tt=tt, r=r):
                    acc, acc2 = carry
                    sl = pl.ds(g * L, L)
                    v = wbuf[s, bt + tt, sl] + pbuf[sp, r, sl]
                    obuf[tt, sl] = v
                    return acc + v, acc2 + v * v

                acc, acc2 = _g1
                meanv = jnp.broadcast_to(jnp.sum(acc) * (1.0 / H), (L,))
                msqv = jnp.broadcast_to(jnp.sum(acc2) * (1.0 / H), (L,))
                varv = msqv - meanv * meanv + 1e-5
                y = plsc.bitcast(
                    jnp.int32(0x5F3759DF) - (plsc.bitcast(varv, jnp.int32) >> 1),
                    jnp.float32,
                )
                y = y * (1.5 - 0.5 * varv * y * y)
                y = y * (1.5 - 0.5 * varv * y * y)
                y = y * (1.5 - 0.5 * varv * y * y)
                mvs.append(meanv)
                rvs.append(y)

            @plsc.parallel_loop(0, GPH, unroll=U2)
            def _g2(g):
                sl = pl.ds(g * L, L)
                gv = gam_v[sl]
                bv = bet_v[sl]
                for tt in range(4):
                    v = obuf[tt, sl]
                    wbuf[s, bt + tt, sl] = (v - mvs[tt]) * rvs[tt] * gv + bv

            return 0

        lax.fori_loop(0, CHUNK // 4, _blk, 0)

    wcopies = {}
    pcopies = {}
    writes = {}
    wcopies[0] = _start_word(0, 0)
    pcopies[0] = _start_pos(0, 0)
    for j in range(NCH):
        s = j % NSLOT
        sp = j % 2
        if j + 1 < NCH:
            if j >= NSLOT - 1:
                writes[j - (NSLOT - 1)].wait()
            wcopies[j + 1] = _start_word(j + 1, (j + 1) % NSLOT)
            pcopies[j + 1] = _start_pos(j + 1, (j + 1) % 2)
        wcopies.pop(j).wait()
        pcopies.pop(j).wait()
        _compute(j, s, sp)
        writes[j] = pltpu.async_copy(
            wbuf.at[s], out_hbm.at[pl.ds(base + j * CHUNK, CHUNK)], semo[s]
        )
    for j in range(NCH - NSLOT, NCH):
        writes[j].wait()


def kernel(arg0_1, arg1_1, arg2_1, arg3_1, arg4_1, arg5_1):
    tok = arg0_1.astype(jnp.int32)
    ptab = _fuse_tables(arg5_1, arg2_1)
    flat = _sc_embed_ln(tok, arg1_1, ptab, arg3_1, arg4_1)
    out = flat.reshape(B, S, H)
    sel = jnp.full((B, S), -0.0, dtype=jnp.float32)
    return (out, sel)
